# baseline (speedup 1.0000x reference)
import functools

import jax
import jax.numpy as jnp
import numpy as np
from jax import lax
from jax.experimental import pallas as pl
from jax.experimental.pallas import tpu as pltpu
from jax.experimental.pallas import tpu_sc as plsc

_N = 262144
_L = 16
_F = 2
_T = 2 ** 19
_MASK = _T - 1
_HIDDEN = 64

_bfac = np.exp((np.log(2048.0) - np.log(16.0)) / (_L - 1))
_RES = [float(r) for r in
        np.floor(16.0 * _bfac ** np.arange(_L)).astype(np.int64)]
_P2 = int(np.uint32(2654435761).astype(np.int32))
_P3 = int(np.uint32(805459861).astype(np.int32))

_NC = 2
_NS = 16
_NW = _NC * _NS
_PPW = _N // _NW
_C = 128
_NCHUNK = _PPW // _C
_NIDX = _L * 8 * _C
_TOTCH = _N // _C

_CORNERS = [(dx, dy, dz) for dx in (0, 1) for dy in (0, 1) for dz in (0, 1)]


def _encode_body(xt_hbm, tbl_hbm, out_hbm, xv, fracv, idxv, rowsv, featsv, sem):
    wid = lax.axis_index("s") * _NC + lax.axis_index("c")
    base_w = wid * _PPW

    def chunk_body(ci, carry):
        base = base_w + ci * _C
        chunkid = wid * _NCHUNK + ci
        for d in range(3):
            pltpu.sync_copy(xt_hbm.at[pl.ds(d * _N + base, _C)], xv.at[d])

        def hash_group(g, carry2):
            off = g * 16
            xs = []
            for d in range(3):
                v = xv[d, pl.ds(off, 16)]
                v = jnp.minimum(jnp.maximum((v + 1.0) * 0.5, 0.0), 1.0 - 1e-6)
                xs.append(v)
            for l in range(_L):
                r = _RES[l]
                px, py, pz = xs[0] * r, xs[1] * r, xs[2] * r
                ix = px.astype(jnp.int32)
                iy = py.astype(jnp.int32)
                iz = pz.astype(jnp.int32)
                fracv[3 * l + 0, pl.ds(off, 16)] = px - ix.astype(jnp.float32)
                fracv[3 * l + 1, pl.ds(off, 16)] = py - iy.astype(jnp.float32)
                fracv[3 * l + 2, pl.ds(off, 16)] = pz - iz.astype(jnp.float32)
                hx = (ix, ix + 1)
                hy = (iy * _P2, (iy + 1) * _P2)
                hz = (iz * _P3, (iz + 1) * _P3)
                for c, (dx, dy, dz) in enumerate(_CORNERS):
                    s = (l * 8 + c) * _C + off
                    e = (((hx[dx] ^ hy[dy] ^ hz[dz]) & _MASK) + l * _T) * 2
                    idxv[pl.ds(s, 16)] = e
                    idxv[pl.ds(_NIDX + s, 16)] = e + 1
            return carry2

        lax.fori_loop(0, _C // 16, hash_group, 0)

        pltpu.async_copy(tbl_hbm.at[idxv], rowsv, sem).wait()

        def interp_group(g, carry2):
            off = g * 16
            for l in range(_L):
                fx = fracv[3 * l + 0, pl.ds(off, 16)]
                fy = fracv[3 * l + 1, pl.ds(off, 16)]
                fz = fracv[3 * l + 2, pl.ds(off, 16)]
                wx = (1.0 - fx, fx)
                wy = (1.0 - fy, fy)
                wz = (1.0 - fz, fz)
                acc0 = jnp.zeros((16,), jnp.float32)
                acc1 = jnp.zeros((16,), jnp.float32)
                for c, (dx, dy, dz) in enumerate(_CORNERS):
                    s = (l * 8 + c) * _C + off
                    f0 = rowsv[pl.ds(s, 16)]
                    f1 = rowsv[pl.ds(_NIDX + s, 16)]
                    w = wx[dx] * wy[dy] * wz[dz]
                    acc0 = acc0 + w * f0
                    acc1 = acc1 + w * f1
                featsv[2 * l, pl.ds(off, 16)] = acc0
                featsv[2 * l + 1, pl.ds(off, 16)] = acc1
            return carry2

        lax.fori_loop(0, _C // 16, interp_group, 0)

        pltpu.sync_copy(featsv, out_hbm.at[chunkid])
        return carry

    lax.fori_loop(0, _NCHUNK, chunk_body, 0)


def _encode(xt, tbl):
    mesh = plsc.VectorSubcoreMesh(core_axis_name="c", subcore_axis_name="s")
    f = pl.kernel(
        _encode_body,
        out_type=jax.ShapeDtypeStruct((_TOTCH, 2 * _L, _C), jnp.float32),
        mesh=mesh,
        scratch_types=[
            pltpu.VMEM((3, _C), jnp.float32),
            pltpu.VMEM((3 * _L, _C), jnp.float32),
            pltpu.VMEM((2 * _NIDX,), jnp.int32),
            pltpu.VMEM((2 * _NIDX,), jnp.float32),
            pltpu.VMEM((2 * _L, _C), jnp.float32),
            pltpu.SemaphoreType.DMA,
        ],
    )
    return f(xt, tbl)


_KB = 16


def _mlp_body(f_ref, w1t_ref, b1_ref, w2_ref, b2_ref, o_ref):
    w1t = w1t_ref[...]
    b1 = b1_ref[...]
    w2 = w2_ref[...]
    b2 = b2_ref[0, 0]
    for j in range(_KB):
        f = f_ref[j]
        h = lax.dot_general(w1t, f, (((1,), (0,)), ((), ())),
                            preferred_element_type=jnp.float32)
        h = jnp.maximum(h + b1, 0.0)
        o_ref[j, 0, :] = jnp.sum(h * w2, axis=0) + b2


def _mlp(feats, W1, b1, W2, b2):
    grid = (_TOTCH // _KB,)
    out = pl.pallas_call(
        _mlp_body,
        grid=grid,
        in_specs=[
            pl.BlockSpec((_KB, 2 * _L, _C), lambda i: (i, 0, 0)),
            pl.BlockSpec((_HIDDEN, 2 * _L), lambda i: (0, 0)),
            pl.BlockSpec((_HIDDEN, 1), lambda i: (0, 0)),
            pl.BlockSpec((_HIDDEN, 1), lambda i: (0, 0)),
            pl.BlockSpec((1, 1), lambda i: (0, 0)),
        ],
        out_specs=pl.BlockSpec((_KB, 1, _C), lambda i: (i, 0, 0)),
        out_shape=jax.ShapeDtypeStruct((_TOTCH, 1, _C), jnp.float32),
    )(feats, W1.T, b1.reshape(_HIDDEN, 1), W2.reshape(_HIDDEN, 1),
      b2.reshape(1, 1))
    return out.reshape(_N)


def kernel(x, tables, W1, b1, W2, b2):
    xt = x.T.reshape(3 * _N)
    tbl = tables.reshape(_L * _T * _F)
    feats = _encode(xt, tbl)
    return _mlp(feats, W1, b1, W2, b2)


# speedup vs baseline: 1.0042x; 1.0042x over previous
import functools

import jax
import jax.numpy as jnp
import numpy as np
from jax import lax
from jax.experimental import pallas as pl
from jax.experimental.pallas import tpu as pltpu
from jax.experimental.pallas import tpu_sc as plsc

_N = 262144
_L = 16
_F = 2
_T = 2 ** 19
_MASK = _T - 1
_HIDDEN = 64

_bfac = np.exp((np.log(2048.0) - np.log(16.0)) / (_L - 1))
_RES = [float(r) for r in
        np.floor(16.0 * _bfac ** np.arange(_L)).astype(np.int64)]
_P2 = int(np.uint32(2654435761).astype(np.int32))
_P3 = int(np.uint32(805459861).astype(np.int32))

_NC = 2
_NS = 16
_NW = _NC * _NS
_PPW = _N // _NW
_C = 128
_M = 4
_SUB = _C // _M
_BPS = 2 * _L * 8 * _SUB
_HBPS = _BPS // 2
_NCHUNK = _PPW // _C
_TOTCH = _N // _C

_CORNERS = [(dx, dy, dz) for dx in (0, 1) for dy in (0, 1) for dz in (0, 1)]

_GDN = lax.GatherDimensionNumbers(offset_dims=(), collapsed_slice_dims=(0,),
                                  start_index_map=(0,))
_SRCBOUNDS = {0: (6, 11), 1: (5, 11), 2: (5, 10)}


def _deinterleave(a, b, c, lane):
    out = []
    for d in range(3):
        lo, hi = _SRCBOUNDS[d]
        pidx = ((lane * 3 + d) & 15)[:, None]
        tk = lambda v: lax.gather(v, pidx, _GDN, (1,),
                                  mode=lax.GatherScatterMode.PROMISE_IN_BOUNDS)
        v = jnp.where(lane < lo, tk(a), jnp.where(lane < hi, tk(b), tk(c)))
        out.append(v)
    return out


def _encode_body(x_hbm, tbl_hbm, out_hbm, xv, fracv, idxs, rows, featsv, sem):
    wid = lax.axis_index("s") * _NC + lax.axis_index("c")
    base_w = wid * _PPW
    lane = lax.iota(jnp.int32, 16)

    def chunk_body(ci, carry):
        base = base_w + ci * _C
        chunkid = wid * _NCHUNK + ci
        pltpu.sync_copy(x_hbm.at[pl.ds(3 * base, 3 * _C)], xv)

        handles = []
        for m in range(_M):
            def hash_group(g, carry2, m=m):
                off = m * _SUB + g * 16
                soff = g * 16
                a = xv[pl.ds(3 * off, 16)]
                b = xv[pl.ds(3 * off + 16, 16)]
                c3 = xv[pl.ds(3 * off + 32, 16)]
                xs = _deinterleave(a, b, c3, lane)
                xs = [jnp.minimum(jnp.maximum((v + 1.0) * 0.5, 0.0),
                                  1.0 - 1e-6) for v in xs]
                for l in range(_L):
                    r = _RES[l]
                    px, py, pz = xs[0] * r, xs[1] * r, xs[2] * r
                    ix = px.astype(jnp.int32)
                    iy = py.astype(jnp.int32)
                    iz = pz.astype(jnp.int32)
                    fracv[3 * l + 0, pl.ds(off, 16)] = px - ix.astype(jnp.float32)
                    fracv[3 * l + 1, pl.ds(off, 16)] = py - iy.astype(jnp.float32)
                    fracv[3 * l + 2, pl.ds(off, 16)] = pz - iz.astype(jnp.float32)
                    hx = (ix, ix + 1)
                    hy = (iy * _P2, (iy + 1) * _P2)
                    hz = (iz * _P3, (iz + 1) * _P3)
                    for c, (dx, dy, dz) in enumerate(_CORNERS):
                        s = (l * 8 + c) * _SUB + soff
                        e = (((hx[dx] ^ hy[dy] ^ hz[dz]) & _MASK) + l * _T) * 2
                        idxs[m][pl.ds(s, 16)] = e
                        idxs[m][pl.ds(_HBPS + s, 16)] = e + 1
                return carry2

            lax.fori_loop(0, _SUB // 16, hash_group, 0)
            handles.append(pltpu.async_copy(tbl_hbm.at[idxs[m]], rows[m], sem))

        for h in handles:
            h.wait()

        for m in range(_M):
            def interp_group(g, carry2, m=m):
                off = m * _SUB + g * 16
                soff = g * 16
                for l in range(_L):
                    fx = fracv[3 * l + 0, pl.ds(off, 16)]
                    fy = fracv[3 * l + 1, pl.ds(off, 16)]
                    fz = fracv[3 * l + 2, pl.ds(off, 16)]
                    wx = (1.0 - fx, fx)
                    wy = (1.0 - fy, fy)
                    wz = (1.0 - fz, fz)
                    acc0 = jnp.zeros((16,), jnp.float32)
                    acc1 = jnp.zeros((16,), jnp.float32)
                    for c, (dx, dy, dz) in enumerate(_CORNERS):
                        s = (l * 8 + c) * _SUB + soff
                        f0 = rows[m][pl.ds(s, 16)]
                        f1 = rows[m][pl.ds(_HBPS + s, 16)]
                        w = wx[dx] * wy[dy] * wz[dz]
                        acc0 = acc0 + w * f0
                        acc1 = acc1 + w * f1
                    featsv[2 * l, pl.ds(off, 16)] = acc0
                    featsv[2 * l + 1, pl.ds(off, 16)] = acc1
                return carry2

            lax.fori_loop(0, _SUB // 16, interp_group, 0)

        pltpu.sync_copy(featsv, out_hbm.at[chunkid])
        return carry

    lax.fori_loop(0, _NCHUNK, chunk_body, 0)


def _encode_entry(x_hbm, tbl_hbm, out_hbm, xv, fracv,
                  i0, i1, i2, i3, r0, r1, r2, r3, featsv, sem):
    _encode_body(x_hbm, tbl_hbm, out_hbm, xv, fracv,
                 [i0, i1, i2, i3], [r0, r1, r2, r3], featsv, sem)


def _encode(xflat, tbl):
    mesh = plsc.VectorSubcoreMesh(core_axis_name="c", subcore_axis_name="s")
    f = pl.kernel(
        _encode_entry,
        out_type=jax.ShapeDtypeStruct((_TOTCH, 2 * _L, _C), jnp.float32),
        mesh=mesh,
        scratch_types=(
            [pltpu.VMEM((3 * _C,), jnp.float32),
             pltpu.VMEM((3 * _L, _C), jnp.float32)]
            + [pltpu.VMEM((_BPS,), jnp.int32) for _ in range(_M)]
            + [pltpu.VMEM((_BPS,), jnp.float32) for _ in range(_M)]
            + [pltpu.VMEM((2 * _L, _C), jnp.float32),
               pltpu.SemaphoreType.DMA]
        ),
    )
    return f(xflat, tbl)


_KB = 16


def _mlp_body(f_ref, w1_ref, b1_ref, w2_ref, b2_ref, o_ref):
    w1 = w1_ref[...]
    b1 = b1_ref[...]
    w2 = w2_ref[...]
    b2 = b2_ref[0, 0]
    fwide = jnp.concatenate([f_ref[j] for j in range(_KB)], axis=1)
    h = lax.dot_general(w1, fwide, (((0,), (0,)), ((), ())),
                        preferred_element_type=jnp.float32)
    h = jnp.maximum(h + b1, 0.0)
    res = jnp.sum(h * w2, axis=0) + b2
    for j in range(_KB):
        o_ref[j, 0, :] = res[j * _C:(j + 1) * _C]


def _mlp(feats, W1, b1, W2, b2):
    grid = (_TOTCH // _KB,)
    out = pl.pallas_call(
        _mlp_body,
        grid=grid,
        in_specs=[
            pl.BlockSpec((_KB, 2 * _L, _C), lambda i: (i, 0, 0)),
            pl.BlockSpec((2 * _L, _HIDDEN), lambda i: (0, 0)),
            pl.BlockSpec((_HIDDEN, 1), lambda i: (0, 0)),
            pl.BlockSpec((_HIDDEN, 1), lambda i: (0, 0)),
            pl.BlockSpec((1, 1), lambda i: (0, 0)),
        ],
        out_specs=pl.BlockSpec((_KB, 1, _C), lambda i: (i, 0, 0)),
        out_shape=jax.ShapeDtypeStruct((_TOTCH, 1, _C), jnp.float32),
    )(feats, W1, b1.reshape(_HIDDEN, 1), W2.reshape(_HIDDEN, 1),
      b2.reshape(1, 1))
    return out.reshape(_N)


def kernel(x, tables, W1, b1, W2, b2):
    xflat = x.reshape(3 * _N)
    tbl = tables.reshape(_L * _T * _F)
    feats = _encode(xflat, tbl)
    return _mlp(feats, W1, b1, W2, b2)


# speedup vs baseline: 4.7288x; 4.7091x over previous
import functools

import jax
import jax.numpy as jnp
import numpy as np
from jax import lax
from jax.experimental import pallas as pl
from jax.experimental.pallas import tpu as pltpu
from jax.experimental.pallas import tpu_sc as plsc

_N = 262144
_L = 16
_F = 2
_T = 2 ** 19
_MASK = _T - 1
_HIDDEN = 64

_bfac = np.exp((np.log(2048.0) - np.log(16.0)) / (_L - 1))
_RES = [float(r) for r in
        np.floor(16.0 * _bfac ** np.arange(_L)).astype(np.int64)]
_P2 = int(np.uint32(2654435761).astype(np.int32))
_P3 = int(np.uint32(805459861).astype(np.int32))

_NC = 2
_NS = 16
_NW = _NC * _NS
_PPW = _N // _NW
_C = 128
_M = 4
_SUB = _C // _M
_BPS = 2 * _L * 8 * _SUB
_HBPS = _BPS // 2
_NCHUNK = _PPW // _C
_TOTCH = _N // _C

_CORNERS = [(dx, dy, dz) for dx in (0, 1) for dy in (0, 1) for dz in (0, 1)]

_GDN = lax.GatherDimensionNumbers(offset_dims=(), collapsed_slice_dims=(0,),
                                  start_index_map=(0,))
_SRCBOUNDS = {0: (6, 11), 1: (5, 11), 2: (5, 10)}


def _deinterleave(a, b, c, lane):
    out = []
    for d in range(3):
        lo, hi = _SRCBOUNDS[d]
        pidx = ((lane * 3 + d) & 15)[:, None]
        tk = lambda v: lax.gather(v, pidx, _GDN, (1,),
                                  mode=lax.GatherScatterMode.PROMISE_IN_BOUNDS)
        v = jnp.where(lane < lo, tk(a), jnp.where(lane < hi, tk(b), tk(c)))
        out.append(v)
    return out


def _encode_body(x_hbm, tbl_hbm, out_hbm, xv, fracv, idxs, rows, featsv, sem):
    wid = lax.axis_index("s") * _NC + lax.axis_index("c")
    base_w = wid * _PPW
    lane = lax.iota(jnp.int32, 16)

    def chunk_body(ci, carry):
        base = base_w + ci * _C
        chunkid = wid * _NCHUNK + ci
        pltpu.sync_copy(x_hbm.at[pl.ds(3 * base, 3 * _C)], xv)

        handles = []
        for m in range(_M):
            def hash_group(g, carry2, m=m):
                off = m * _SUB + g * 16
                soff = g * 16
                a = xv[pl.ds(3 * off, 16)]
                b = xv[pl.ds(3 * off + 16, 16)]
                c3 = xv[pl.ds(3 * off + 32, 16)]
                xs = _deinterleave(a, b, c3, lane)
                xs = [jnp.minimum(jnp.maximum((v + 1.0) * 0.5, 0.0),
                                  1.0 - 1e-6) for v in xs]
                for l in range(_L):
                    r = _RES[l]
                    px, py, pz = xs[0] * r, xs[1] * r, xs[2] * r
                    ix = px.astype(jnp.int32)
                    iy = py.astype(jnp.int32)
                    iz = pz.astype(jnp.int32)
                    fracv[3 * l + 0, pl.ds(off, 16)] = px - ix.astype(jnp.float32)
                    fracv[3 * l + 1, pl.ds(off, 16)] = py - iy.astype(jnp.float32)
                    fracv[3 * l + 2, pl.ds(off, 16)] = pz - iz.astype(jnp.float32)
                    hx = (ix, ix + 1)
                    hy = (iy * _P2, (iy + 1) * _P2)
                    hz = (iz * _P3, (iz + 1) * _P3)
                    for c, (dx, dy, dz) in enumerate(_CORNERS):
                        s = (l * 8 + c) * _SUB + soff
                        t = (hx[dx] ^ hy[dy] ^ hz[dz]) & _MASK
                        e = (((t & 0x7FF80) << 1) + (t & 127)) + l * (2 * _T)
                        idxs[m][pl.ds(s, 16)] = e
                        idxs[m][pl.ds(_HBPS + s, 16)] = e + 128
                return carry2

            lax.fori_loop(0, _SUB // 16, hash_group, 0)
            handles.append(pltpu.async_copy(tbl_hbm.at[idxs[m]], rows[m], sem))

        for h in handles:
            h.wait()

        for m in range(_M):
            def interp_group(g, carry2, m=m):
                off = m * _SUB + g * 16
                soff = g * 16
                for l in range(_L):
                    fx = fracv[3 * l + 0, pl.ds(off, 16)]
                    fy = fracv[3 * l + 1, pl.ds(off, 16)]
                    fz = fracv[3 * l + 2, pl.ds(off, 16)]
                    wx = (1.0 - fx, fx)
                    wy = (1.0 - fy, fy)
                    wz = (1.0 - fz, fz)
                    acc0 = jnp.zeros((16,), jnp.float32)
                    acc1 = jnp.zeros((16,), jnp.float32)
                    for c, (dx, dy, dz) in enumerate(_CORNERS):
                        s = (l * 8 + c) * _SUB + soff
                        f0 = rows[m][pl.ds(s, 16)]
                        f1 = rows[m][pl.ds(_HBPS + s, 16)]
                        w = wx[dx] * wy[dy] * wz[dz]
                        acc0 = acc0 + w * f0
                        acc1 = acc1 + w * f1
                    featsv[2 * l, pl.ds(off, 16)] = acc0
                    featsv[2 * l + 1, pl.ds(off, 16)] = acc1
                return carry2

            lax.fori_loop(0, _SUB // 16, interp_group, 0)

        pltpu.sync_copy(featsv, out_hbm.at[chunkid])
        return carry

    lax.fori_loop(0, _NCHUNK, chunk_body, 0)


def _encode_entry(x_hbm, tbl_hbm, out_hbm, xv, fracv,
                  i0, i1, i2, i3, r0, r1, r2, r3, featsv, sem):
    _encode_body(x_hbm, tbl_hbm, out_hbm, xv, fracv,
                 [i0, i1, i2, i3], [r0, r1, r2, r3], featsv, sem)


def _encode(xflat, tbl):
    mesh = plsc.VectorSubcoreMesh(core_axis_name="c", subcore_axis_name="s")
    f = pl.kernel(
        _encode_entry,
        out_type=jax.ShapeDtypeStruct((_TOTCH, 2 * _L, _C), jnp.float32),
        mesh=mesh,
        compiler_params=pltpu.CompilerParams(needs_layout_passes=False),
        scratch_types=(
            [pltpu.VMEM((3 * _C,), jnp.float32),
             pltpu.VMEM((3 * _L, _C), jnp.float32)]
            + [pltpu.VMEM((_BPS,), jnp.int32) for _ in range(_M)]
            + [pltpu.VMEM((_BPS,), jnp.float32) for _ in range(_M)]
            + [pltpu.VMEM((2 * _L, _C), jnp.float32),
               pltpu.SemaphoreType.DMA]
        ),
    )
    return f(xflat, tbl)


_KB = 16


def _mlp_body(f_ref, w1_ref, b1_ref, w2_ref, b2_ref, o_ref):
    w1 = w1_ref[...]
    b1 = b1_ref[...]
    w2 = w2_ref[...]
    b2 = b2_ref[0, 0]
    fwide = jnp.concatenate([f_ref[j] for j in range(_KB)], axis=1)
    h = lax.dot_general(w1, fwide, (((0,), (0,)), ((), ())),
                        preferred_element_type=jnp.float32)
    h = jnp.maximum(h + b1, 0.0)
    res = jnp.sum(h * w2, axis=0) + b2
    for j in range(_KB):
        o_ref[j, 0, :] = res[j * _C:(j + 1) * _C]


def _mlp(feats, W1, b1, W2, b2):
    grid = (_TOTCH // _KB,)
    out = pl.pallas_call(
        _mlp_body,
        grid=grid,
        in_specs=[
            pl.BlockSpec((_KB, 2 * _L, _C), lambda i: (i, 0, 0)),
            pl.BlockSpec((2 * _L, _HIDDEN), lambda i: (0, 0)),
            pl.BlockSpec((_HIDDEN, 1), lambda i: (0, 0)),
            pl.BlockSpec((_HIDDEN, 1), lambda i: (0, 0)),
            pl.BlockSpec((1, 1), lambda i: (0, 0)),
        ],
        out_specs=pl.BlockSpec((_KB, 1, _C), lambda i: (i, 0, 0)),
        out_shape=jax.ShapeDtypeStruct((_TOTCH, 1, _C), jnp.float32),
    )(feats, W1, b1.reshape(_HIDDEN, 1), W2.reshape(_HIDDEN, 1),
      b2.reshape(1, 1))
    return out.reshape(_N)


def kernel(x, tables, W1, b1, W2, b2):
    xflat = x.reshape(3 * _N)
    tbl = tables.reshape(_L, _T // 128, 128, _F).transpose(0, 1, 3, 2)
    tbl = tbl.reshape(_L * _T * _F)
    feats = _encode(xflat, tbl)
    return _mlp(feats, W1, b1, W2, b2)


# speedup vs baseline: 7.9795x; 1.6874x over previous
import functools

import jax
import jax.numpy as jnp
import numpy as np
from jax import lax
from jax.experimental import pallas as pl
from jax.experimental.pallas import tpu as pltpu
from jax.experimental.pallas import tpu_sc as plsc

_N = 262144
_L = 16
_F = 2
_T = 2 ** 19
_MASK = _T - 1
_HIDDEN = 64

_bfac = np.exp((np.log(2048.0) - np.log(16.0)) / (_L - 1))
_RES = [float(r) for r in
        np.floor(16.0 * _bfac ** np.arange(_L)).astype(np.int64)]
_P2 = int(np.uint32(2654435761).astype(np.int32))
_P3 = int(np.uint32(805459861).astype(np.int32))

_NC = 2
_NS = 16
_NW = _NC * _NS
_PPW = _N // _NW
_C = 128
_M = 4
_SUB = _C // _M
_BPS = _L * 8 * _SUB
_NCHUNK = _PPW // _C
_TOTCH = _N // _C

_CORNERS = [(dx, dy, dz) for dx in (0, 1) for dy in (0, 1) for dz in (0, 1)]

_GDN = lax.GatherDimensionNumbers(offset_dims=(), collapsed_slice_dims=(0,),
                                  start_index_map=(0,))
_SRCBOUNDS = {0: (6, 11), 1: (5, 11), 2: (5, 10)}


def _deinterleave(a, b, c, lane):
    out = []
    for d in range(3):
        lo, hi = _SRCBOUNDS[d]
        pidx = ((lane * 3 + d) & 15)[:, None]
        tk = lambda v: lax.gather(v, pidx, _GDN, (1,),
                                  mode=lax.GatherScatterMode.PROMISE_IN_BOUNDS)
        v = jnp.where(lane < lo, tk(a), jnp.where(lane < hi, tk(b), tk(c)))
        out.append(v)
    return out


def _encode_body(x_hbm, tbl_hbm, out_hbm, xv, fracv, idxs, rows, featsv, sem):
    wid = lax.axis_index("s") * _NC + lax.axis_index("c")
    base_w = wid * _PPW
    lane = lax.iota(jnp.int32, 16)

    def chunk_body(ci, carry):
        base = base_w + ci * _C
        chunkid = wid * _NCHUNK + ci
        pltpu.sync_copy(x_hbm.at[pl.ds(3 * base, 3 * _C)], xv)

        handles = []
        for m in range(_M):
            def hash_group(g, carry2, m=m):
                off = m * _SUB + g * 16
                soff = g * 16
                a = xv[pl.ds(3 * off, 16)]
                b = xv[pl.ds(3 * off + 16, 16)]
                c3 = xv[pl.ds(3 * off + 32, 16)]
                xs = _deinterleave(a, b, c3, lane)
                xs = [jnp.minimum(jnp.maximum((v + 1.0) * 0.5, 0.0),
                                  1.0 - 1e-6) for v in xs]
                for l in range(_L):
                    r = _RES[l]
                    px, py, pz = xs[0] * r, xs[1] * r, xs[2] * r
                    ix = px.astype(jnp.int32)
                    iy = py.astype(jnp.int32)
                    iz = pz.astype(jnp.int32)
                    fracv[3 * l + 0, pl.ds(off, 16)] = px - ix.astype(jnp.float32)
                    fracv[3 * l + 1, pl.ds(off, 16)] = py - iy.astype(jnp.float32)
                    fracv[3 * l + 2, pl.ds(off, 16)] = pz - iz.astype(jnp.float32)
                    hx = (ix, ix + 1)
                    hy = (iy * _P2, (iy + 1) * _P2)
                    hz = (iz * _P3, (iz + 1) * _P3)
                    for c, (dx, dy, dz) in enumerate(_CORNERS):
                        s = (l * 8 + c) * _SUB + soff
                        e = ((hx[dx] ^ hy[dy] ^ hz[dz]) & _MASK) + l * _T
                        idxs[m][pl.ds(s, 16)] = e
                return carry2

            lax.fori_loop(0, _SUB // 16, hash_group, 0)
            handles.append(pltpu.async_copy(tbl_hbm.at[idxs[m]], rows[m], sem))

        for h in handles:
            h.wait()

        for m in range(_M):
            def interp_group(g, carry2, m=m):
                off = m * _SUB + g * 16
                soff = g * 16
                for l in range(_L):
                    fx = fracv[3 * l + 0, pl.ds(off, 16)]
                    fy = fracv[3 * l + 1, pl.ds(off, 16)]
                    fz = fracv[3 * l + 2, pl.ds(off, 16)]
                    wx = (1.0 - fx, fx)
                    wy = (1.0 - fy, fy)
                    wz = (1.0 - fz, fz)
                    acc0 = jnp.zeros((16,), jnp.float32)
                    acc1 = jnp.zeros((16,), jnp.float32)
                    for c, (dx, dy, dz) in enumerate(_CORNERS):
                        s = (l * 8 + c) * _SUB + soff
                        v = rows[m][pl.ds(s, 16)]
                        f0 = plsc.bitcast(v << 16, jnp.float32)
                        f1 = plsc.bitcast(v & jnp.int32(-65536), jnp.float32)
                        w = wx[dx] * wy[dy] * wz[dz]
                        acc0 = acc0 + w * f0
                        acc1 = acc1 + w * f1
                    featsv[2 * l, pl.ds(off, 16)] = acc0
                    featsv[2 * l + 1, pl.ds(off, 16)] = acc1
                return carry2

            lax.fori_loop(0, _SUB // 16, interp_group, 0)

        pltpu.sync_copy(featsv, out_hbm.at[chunkid])
        return carry

    lax.fori_loop(0, _NCHUNK, chunk_body, 0)


def _encode_entry(x_hbm, tbl_hbm, out_hbm, xv, fracv,
                  i0, i1, i2, i3, r0, r1, r2, r3, featsv, sem):
    _encode_body(x_hbm, tbl_hbm, out_hbm, xv, fracv,
                 [i0, i1, i2, i3], [r0, r1, r2, r3], featsv, sem)


def _encode(xflat, tbl):
    mesh = plsc.VectorSubcoreMesh(core_axis_name="c", subcore_axis_name="s")
    f = pl.kernel(
        _encode_entry,
        out_type=jax.ShapeDtypeStruct((_TOTCH, 2 * _L, _C), jnp.float32),
        mesh=mesh,
        compiler_params=pltpu.CompilerParams(needs_layout_passes=False),
        scratch_types=(
            [pltpu.VMEM((3 * _C,), jnp.float32),
             pltpu.VMEM((3 * _L, _C), jnp.float32)]
            + [pltpu.VMEM((_BPS,), jnp.int32) for _ in range(_M)]
            + [pltpu.VMEM((_BPS,), jnp.int32) for _ in range(_M)]
            + [pltpu.VMEM((2 * _L, _C), jnp.float32),
               pltpu.SemaphoreType.DMA]
        ),
    )
    return f(xflat, tbl)


_PBLK = 64
_NBLK = _L * (_T // 128)
_BPW = _NBLK // _NW
_NPB = _BPW // _PBLK


def _pack_body(tblv_hbm, out_hbm, inv, outv, sem):
    wid = lax.axis_index("s") * _NC + lax.axis_index("c")
    b0 = wid * _BPW

    def it(k, carry):
        blk = b0 + k * _PBLK
        pltpu.sync_copy(tblv_hbm.at[pl.ds(blk * 256, _PBLK * 256)], inv)

        def grp(j, c2):
            jb = j >> 3
            g = j & 7
            off_in = jb * 256 + g * 16
            f0 = inv[pl.ds(off_in, 16)]
            f1 = inv[pl.ds(off_in + 128, 16)]
            pk = plsc.pack(f0, f1, format=plsc.PackFormat.INTERLEAVED)
            outv[pl.ds(jb * 128 + g * 16, 16)] = plsc.bitcast(pk, jnp.int32)
            return c2

        lax.fori_loop(0, _PBLK * 8, grp, 0)
        pltpu.sync_copy(outv, out_hbm.at[pl.ds(blk * 128, _PBLK * 128)])
        return carry

    lax.fori_loop(0, _NPB, it, 0)


def _pack(tblv):
    mesh = plsc.VectorSubcoreMesh(core_axis_name="c", subcore_axis_name="s")
    f = pl.kernel(
        _pack_body,
        out_type=jax.ShapeDtypeStruct((_L * _T,), jnp.int32),
        mesh=mesh,
        compiler_params=pltpu.CompilerParams(needs_layout_passes=False),
        scratch_types=[
            pltpu.VMEM((_PBLK * 256,), jnp.float32),
            pltpu.VMEM((_PBLK * 128,), jnp.int32),
            pltpu.SemaphoreType.DMA,
        ],
    )
    return f(tblv)


_KB = 16


def _mlp_body(f_ref, w1_ref, b1_ref, w2_ref, b2_ref, o_ref):
    w1 = w1_ref[...]
    b1 = b1_ref[...]
    w2 = w2_ref[...]
    b2 = b2_ref[0, 0]
    fwide = jnp.concatenate([f_ref[j] for j in range(_KB)], axis=1)
    h = lax.dot_general(w1, fwide, (((0,), (0,)), ((), ())),
                        preferred_element_type=jnp.float32)
    h = jnp.maximum(h + b1, 0.0)
    res = jnp.sum(h * w2, axis=0) + b2
    for j in range(_KB):
        o_ref[j, 0, :] = res[j * _C:(j + 1) * _C]


def _mlp(feats, W1, b1, W2, b2):
    grid = (_TOTCH // _KB,)
    out = pl.pallas_call(
        _mlp_body,
        grid=grid,
        in_specs=[
            pl.BlockSpec((_KB, 2 * _L, _C), lambda i: (i, 0, 0)),
            pl.BlockSpec((2 * _L, _HIDDEN), lambda i: (0, 0)),
            pl.BlockSpec((_HIDDEN, 1), lambda i: (0, 0)),
            pl.BlockSpec((_HIDDEN, 1), lambda i: (0, 0)),
            pl.BlockSpec((1, 1), lambda i: (0, 0)),
        ],
        out_specs=pl.BlockSpec((_KB, 1, _C), lambda i: (i, 0, 0)),
        out_shape=jax.ShapeDtypeStruct((_TOTCH, 1, _C), jnp.float32),
    )(feats, W1, b1.reshape(_HIDDEN, 1), W2.reshape(_HIDDEN, 1),
      b2.reshape(1, 1))
    return out.reshape(_N)


def kernel(x, tables, W1, b1, W2, b2):
    xflat = x.reshape(3 * _N)
    tblv = tables.reshape(_L, _T // 128, 128, _F).transpose(0, 1, 3, 2)
    tblv = tblv.reshape(_L * _T * _F)
    tbl = _pack(tblv)
    feats = _encode(xflat, tbl)
    return _mlp(feats, W1, b1, W2, b2)


# speedup vs baseline: 8.5819x; 1.0755x over previous
import functools

import jax
import jax.numpy as jnp
import numpy as np
from jax import lax
from jax.experimental import pallas as pl
from jax.experimental.pallas import tpu as pltpu
from jax.experimental.pallas import tpu_sc as plsc

_N = 262144
_L = 16
_F = 2
_T = 2 ** 19
_MASK = _T - 1
_HIDDEN = 64

_bfac = np.exp((np.log(2048.0) - np.log(16.0)) / (_L - 1))
_RES = [float(r) for r in
        np.floor(16.0 * _bfac ** np.arange(_L)).astype(np.int64)]
_P2 = int(np.uint32(2654435761).astype(np.int32))
_P3 = int(np.uint32(805459861).astype(np.int32))

_NC = 2
_NS = 16
_NW = _NC * _NS
_PPW = _N // _NW
_C = 128
_M = 4
_SUB = _C // _M
_BPS = _L * 8 * _SUB
_NCHUNK = _PPW // _C
_TOTCH = _N // _C

_CORNERS = [(dx, dy, dz) for dx in (0, 1) for dy in (0, 1) for dz in (0, 1)]

_GDN = lax.GatherDimensionNumbers(offset_dims=(), collapsed_slice_dims=(0,),
                                  start_index_map=(0,))
_SRCBOUNDS = {0: (6, 11), 1: (5, 11), 2: (5, 10)}


def _deinterleave(a, b, c, lane):
    out = []
    for d in range(3):
        lo, hi = _SRCBOUNDS[d]
        pidx = ((lane * 3 + d) & 15)[:, None]
        tk = lambda v: lax.gather(v, pidx, _GDN, (1,),
                                  mode=lax.GatherScatterMode.PROMISE_IN_BOUNDS)
        v = jnp.where(lane < lo, tk(a), jnp.where(lane < hi, tk(b), tk(c)))
        out.append(v)
    return out


def _encode_body(x_hbm, tbl_hbm, out_hbm, xv, fracvs, idxs, rows,
                 featsv, sems):
    wid = lax.axis_index("s") * _NC + lax.axis_index("c")
    base_w = wid * _PPW
    lane = lax.iota(jnp.int32, 16)

    def stage_a(ci, par):
        base = base_w + ci * _C
        pltpu.sync_copy(x_hbm.at[pl.ds(3 * base, 3 * _C)], xv)
        fracv = fracvs[par]
        for m in range(_M):
            def hash_group(g, carry2, m=m):
                off = m * _SUB + g * 16
                soff = g * 16
                a = xv[pl.ds(3 * off, 16)]
                b = xv[pl.ds(3 * off + 16, 16)]
                c3 = xv[pl.ds(3 * off + 32, 16)]
                xs = _deinterleave(a, b, c3, lane)
                xs = [jnp.minimum(jnp.maximum((v + 1.0) * 0.5, 0.0),
                                  1.0 - 1e-6) for v in xs]
                for l in range(_L):
                    r = _RES[l]
                    px, py, pz = xs[0] * r, xs[1] * r, xs[2] * r
                    ix = px.astype(jnp.int32)
                    iy = py.astype(jnp.int32)
                    iz = pz.astype(jnp.int32)
                    fracv[3 * l + 0, pl.ds(off, 16)] = px - ix.astype(jnp.float32)
                    fracv[3 * l + 1, pl.ds(off, 16)] = py - iy.astype(jnp.float32)
                    fracv[3 * l + 2, pl.ds(off, 16)] = pz - iz.astype(jnp.float32)
                    hx = (ix, ix + 1)
                    hy = (iy * _P2, (iy + 1) * _P2)
                    hz = (iz * _P3, (iz + 1) * _P3)
                    for c, (dx, dy, dz) in enumerate(_CORNERS):
                        s = (l * 8 + c) * _SUB + soff
                        e = ((hx[dx] ^ hy[dy] ^ hz[dz]) & _MASK) + l * _T
                        idxs[par * _M + m][pl.ds(s, 16)] = e
                return carry2

            lax.fori_loop(0, _SUB // 16, hash_group, 0)
            pltpu.async_copy(tbl_hbm.at[idxs[par * _M + m]],
                             rows[par * _M + m], sems[par])

    def stage_wait(par):
        for m in range(_M):
            pltpu.make_async_copy(tbl_hbm.at[idxs[par * _M + m]],
                                  rows[par * _M + m], sems[par]).wait()

    def stage_c(ci, par):
        base = base_w + ci * _C
        chunkid = wid * _NCHUNK + ci
        fracv = fracvs[par]
        for m in range(_M):
            def interp_group(g, carry2, m=m):
                off = m * _SUB + g * 16
                soff = g * 16
                for l in range(_L):
                    fx = fracv[3 * l + 0, pl.ds(off, 16)]
                    fy = fracv[3 * l + 1, pl.ds(off, 16)]
                    fz = fracv[3 * l + 2, pl.ds(off, 16)]
                    wx = (1.0 - fx, fx)
                    wy = (1.0 - fy, fy)
                    wz = (1.0 - fz, fz)
                    acc0 = jnp.zeros((16,), jnp.float32)
                    acc1 = jnp.zeros((16,), jnp.float32)
                    for c, (dx, dy, dz) in enumerate(_CORNERS):
                        s = (l * 8 + c) * _SUB + soff
                        v = rows[par * _M + m][pl.ds(s, 16)]
                        f0 = plsc.bitcast(v << 16, jnp.float32)
                        f1 = plsc.bitcast(v & jnp.int32(-65536), jnp.float32)
                        w = wx[dx] * wy[dy] * wz[dz]
                        acc0 = acc0 + w * f0
                        acc1 = acc1 + w * f1
                    featsv[2 * l, pl.ds(off, 16)] = acc0
                    featsv[2 * l + 1, pl.ds(off, 16)] = acc1
                return carry2

            lax.fori_loop(0, _SUB // 16, interp_group, 0)

        pltpu.sync_copy(featsv, out_hbm.at[chunkid])

    stage_a(0, 0)

    def body(k2, carry):
        c0 = 2 * k2
        stage_a(c0 + 1, 1)
        stage_wait(0)
        stage_c(c0, 0)

        @pl.when(c0 + 2 < _NCHUNK)
        def _prefetch():
            stage_a(c0 + 2, 0)

        stage_wait(1)
        stage_c(c0 + 1, 1)
        return carry

    lax.fori_loop(0, _NCHUNK // 2, body, 0)


def _encode_entry(x_hbm, tbl_hbm, out_hbm, xv, f0v, f1v,
                  i0, i1, i2, i3, i4, i5, i6, i7,
                  r0, r1, r2, r3, r4, r5, r6, r7, featsv, semA, semB):
    _encode_body(x_hbm, tbl_hbm, out_hbm, xv, [f0v, f1v],
                 [i0, i1, i2, i3, i4, i5, i6, i7],
                 [r0, r1, r2, r3, r4, r5, r6, r7], featsv, [semA, semB])


def _encode(xflat, tbl):
    mesh = plsc.VectorSubcoreMesh(core_axis_name="c", subcore_axis_name="s")
    f = pl.kernel(
        _encode_entry,
        out_type=jax.ShapeDtypeStruct((_TOTCH, 2 * _L, _C), jnp.float32),
        mesh=mesh,
        compiler_params=pltpu.CompilerParams(needs_layout_passes=False),
        scratch_types=(
            [pltpu.VMEM((3 * _C,), jnp.float32)]
            + [pltpu.VMEM((3 * _L, _C), jnp.float32) for _ in range(2)]
            + [pltpu.VMEM((_BPS,), jnp.int32) for _ in range(2 * _M)]
            + [pltpu.VMEM((_BPS,), jnp.int32) for _ in range(2 * _M)]
            + [pltpu.VMEM((2 * _L, _C), jnp.float32),
               pltpu.SemaphoreType.DMA, pltpu.SemaphoreType.DMA]
        ),
    )
    return f(xflat, tbl)


_PBLK = 64
_NBLK = _L * (_T // 128)
_BPW = _NBLK // _NW
_NPB = _BPW // _PBLK


def _pack_body(tblv_hbm, out_hbm, inv, outv, sem):
    wid = lax.axis_index("s") * _NC + lax.axis_index("c")
    b0 = wid * _BPW

    def it(k, carry):
        blk = b0 + k * _PBLK
        pltpu.sync_copy(tblv_hbm.at[pl.ds(blk * 256, _PBLK * 256)], inv)

        def grp(j, c2):
            jb = j >> 3
            g = j & 7
            off_in = jb * 256 + g * 16
            f0 = inv[pl.ds(off_in, 16)]
            f1 = inv[pl.ds(off_in + 128, 16)]
            pk = plsc.pack(f0, f1, format=plsc.PackFormat.INTERLEAVED)
            outv[pl.ds(jb * 128 + g * 16, 16)] = plsc.bitcast(pk, jnp.int32)
            return c2

        lax.fori_loop(0, _PBLK * 8, grp, 0)
        pltpu.sync_copy(outv, out_hbm.at[pl.ds(blk * 128, _PBLK * 128)])
        return carry

    lax.fori_loop(0, _NPB, it, 0)


def _pack(tblv):
    mesh = plsc.VectorSubcoreMesh(core_axis_name="c", subcore_axis_name="s")
    f = pl.kernel(
        _pack_body,
        out_type=jax.ShapeDtypeStruct((_L * _T,), jnp.int32),
        mesh=mesh,
        compiler_params=pltpu.CompilerParams(needs_layout_passes=False),
        scratch_types=[
            pltpu.VMEM((_PBLK * 256,), jnp.float32),
            pltpu.VMEM((_PBLK * 128,), jnp.int32),
            pltpu.SemaphoreType.DMA,
        ],
    )
    return f(tblv)


_KB = 16


def _mlp_body(f_ref, w1_ref, b1_ref, w2_ref, b2_ref, o_ref):
    w1 = w1_ref[...]
    b1 = b1_ref[...]
    w2 = w2_ref[...]
    b2 = b2_ref[0, 0]
    fwide = jnp.concatenate([f_ref[j] for j in range(_KB)], axis=1)
    h = lax.dot_general(w1, fwide, (((0,), (0,)), ((), ())),
                        preferred_element_type=jnp.float32)
    h = jnp.maximum(h + b1, 0.0)
    res = jnp.sum(h * w2, axis=0) + b2
    for j in range(_KB):
        o_ref[j, 0, :] = res[j * _C:(j + 1) * _C]


def _mlp(feats, W1, b1, W2, b2):
    grid = (_TOTCH // _KB,)
    out = pl.pallas_call(
        _mlp_body,
        grid=grid,
        in_specs=[
            pl.BlockSpec((_KB, 2 * _L, _C), lambda i: (i, 0, 0)),
            pl.BlockSpec((2 * _L, _HIDDEN), lambda i: (0, 0)),
            pl.BlockSpec((_HIDDEN, 1), lambda i: (0, 0)),
            pl.BlockSpec((_HIDDEN, 1), lambda i: (0, 0)),
            pl.BlockSpec((1, 1), lambda i: (0, 0)),
        ],
        out_specs=pl.BlockSpec((_KB, 1, _C), lambda i: (i, 0, 0)),
        out_shape=jax.ShapeDtypeStruct((_TOTCH, 1, _C), jnp.float32),
    )(feats, W1, b1.reshape(_HIDDEN, 1), W2.reshape(_HIDDEN, 1),
      b2.reshape(1, 1))
    return out.reshape(_N)


def kernel(x, tables, W1, b1, W2, b2):
    xflat = x.reshape(3 * _N)
    tblv = tables.reshape(_L, _T // 128, 128, _F).transpose(0, 1, 3, 2)
    tblv = tblv.reshape(_L * _T * _F)
    tbl = _pack(tblv)
    feats = _encode(xflat, tbl)
    return _mlp(feats, W1, b1, W2, b2)


# speedup vs baseline: 8.6514x; 1.0081x over previous
import functools

import jax
import jax.numpy as jnp
import numpy as np
from jax import lax
from jax.experimental import pallas as pl
from jax.experimental.pallas import tpu as pltpu
from jax.experimental.pallas import tpu_sc as plsc

_N = 262144
_L = 16
_F = 2
_T = 2 ** 19
_MASK = _T - 1
_HIDDEN = 64

_bfac = np.exp((np.log(2048.0) - np.log(16.0)) / (_L - 1))
_RES = [float(r) for r in
        np.floor(16.0 * _bfac ** np.arange(_L)).astype(np.int64)]
_P2 = int(np.uint32(2654435761).astype(np.int32))
_P3 = int(np.uint32(805459861).astype(np.int32))

_NC = 2
_NS = 16
_NW = _NC * _NS
_PPW = _N // 2 // _NW
_C = 128
_M = 4
_SUB = _C // _M
_BPS = _L * 8 * _SUB
_NCHUNK = _PPW // _C
_TOTCH = _N // 2 // _C

_CORNERS = [(dx, dy, dz) for dx in (0, 1) for dy in (0, 1) for dz in (0, 1)]

_GDN = lax.GatherDimensionNumbers(offset_dims=(), collapsed_slice_dims=(0,),
                                  start_index_map=(0,))
_SRCBOUNDS = {0: (6, 11), 1: (5, 11), 2: (5, 10)}


def _deinterleave(a, b, c, lane):
    out = []
    for d in range(3):
        lo, hi = _SRCBOUNDS[d]
        pidx = ((lane * 3 + d) & 15)[:, None]
        tk = lambda v: lax.gather(v, pidx, _GDN, (1,),
                                  mode=lax.GatherScatterMode.PROMISE_IN_BOUNDS)
        v = jnp.where(lane < lo, tk(a), jnp.where(lane < hi, tk(b), tk(c)))
        out.append(v)
    return out


def _encode_body(half, x_hbm, tbl_hbm, out_hbm, xv, fracvs, idxs, rows,
                 featsv, sems):
    wid = lax.axis_index("s") * _NC + lax.axis_index("c")
    base_w = half * (_N // 2) + wid * _PPW
    lane = lax.iota(jnp.int32, 16)

    def stage_a(ci, par):
        base = base_w + ci * _C
        pltpu.sync_copy(x_hbm.at[pl.ds(3 * base, 3 * _C)], xv)
        fracv = fracvs[par]
        for m in range(_M):
            def hash_group(g, carry2, m=m):
                off = m * _SUB + g * 16
                soff = g * 16
                a = xv[pl.ds(3 * off, 16)]
                b = xv[pl.ds(3 * off + 16, 16)]
                c3 = xv[pl.ds(3 * off + 32, 16)]
                xs = _deinterleave(a, b, c3, lane)
                xs = [jnp.minimum(jnp.maximum((v + 1.0) * 0.5, 0.0),
                                  1.0 - 1e-6) for v in xs]
                for l in range(_L):
                    r = _RES[l]
                    px, py, pz = xs[0] * r, xs[1] * r, xs[2] * r
                    ix = px.astype(jnp.int32)
                    iy = py.astype(jnp.int32)
                    iz = pz.astype(jnp.int32)
                    fracv[3 * l + 0, pl.ds(off, 16)] = px - ix.astype(jnp.float32)
                    fracv[3 * l + 1, pl.ds(off, 16)] = py - iy.astype(jnp.float32)
                    fracv[3 * l + 2, pl.ds(off, 16)] = pz - iz.astype(jnp.float32)
                    hx = (ix, ix + 1)
                    hy = (iy * _P2, (iy + 1) * _P2)
                    hz = (iz * _P3, (iz + 1) * _P3)
                    for c, (dx, dy, dz) in enumerate(_CORNERS):
                        s = (l * 8 + c) * _SUB + soff
                        e = ((hx[dx] ^ hy[dy] ^ hz[dz]) & _MASK) + l * _T
                        idxs[par * _M + m][pl.ds(s, 16)] = e
                return carry2

            lax.fori_loop(0, _SUB // 16, hash_group, 0)
            pltpu.async_copy(tbl_hbm.at[idxs[par * _M + m]],
                             rows[par * _M + m], sems[par])

    def stage_wait(par):
        for m in range(_M):
            pltpu.make_async_copy(tbl_hbm.at[idxs[par * _M + m]],
                                  rows[par * _M + m], sems[par]).wait()

    def stage_c(ci, par):
        base = base_w + ci * _C
        chunkid = wid * _NCHUNK + ci
        fracv = fracvs[par]
        for m in range(_M):
            def interp_group(g, carry2, m=m):
                off = m * _SUB + g * 16
                soff = g * 16
                for l in range(_L):
                    fx = fracv[3 * l + 0, pl.ds(off, 16)]
                    fy = fracv[3 * l + 1, pl.ds(off, 16)]
                    fz = fracv[3 * l + 2, pl.ds(off, 16)]
                    wx = (1.0 - fx, fx)
                    wy = (1.0 - fy, fy)
                    wz = (1.0 - fz, fz)
                    acc0 = jnp.zeros((16,), jnp.float32)
                    acc1 = jnp.zeros((16,), jnp.float32)
                    for c, (dx, dy, dz) in enumerate(_CORNERS):
                        s = (l * 8 + c) * _SUB + soff
                        v = rows[par * _M + m][pl.ds(s, 16)]
                        f0 = plsc.bitcast(v << 16, jnp.float32)
                        f1 = plsc.bitcast(v & jnp.int32(-65536), jnp.float32)
                        w = wx[dx] * wy[dy] * wz[dz]
                        acc0 = acc0 + w * f0
                        acc1 = acc1 + w * f1
                    featsv[2 * l, pl.ds(off, 16)] = acc0
                    featsv[2 * l + 1, pl.ds(off, 16)] = acc1
                return carry2

            lax.fori_loop(0, _SUB // 16, interp_group, 0)

        pltpu.sync_copy(featsv, out_hbm.at[chunkid])

    stage_a(0, 0)

    def body(k2, carry):
        c0 = 2 * k2
        stage_a(c0 + 1, 1)
        stage_wait(0)
        stage_c(c0, 0)

        @pl.when(c0 + 2 < _NCHUNK)
        def _prefetch():
            stage_a(c0 + 2, 0)

        stage_wait(1)
        stage_c(c0 + 1, 1)
        return carry

    lax.fori_loop(0, _NCHUNK // 2, body, 0)


def _make_encode_entry(half):
    def _encode_entry(x_hbm, tbl_hbm, out_hbm, xv, f0v, f1v, *rest):
        idxs = list(rest[:2 * _M])
        rows = list(rest[2 * _M:4 * _M])
        featsv = rest[4 * _M]
        sems = list(rest[4 * _M + 1:4 * _M + 3])
        _encode_body(half, x_hbm, tbl_hbm, out_hbm, xv, [f0v, f1v], idxs,
                     rows, featsv, sems)
    return _encode_entry


def _encode(xflat, tbl, half):
    mesh = plsc.VectorSubcoreMesh(core_axis_name="c", subcore_axis_name="s")
    f = pl.kernel(
        _make_encode_entry(half),
        out_type=jax.ShapeDtypeStruct((_TOTCH, 2 * _L, _C), jnp.float32),
        mesh=mesh,
        compiler_params=pltpu.CompilerParams(needs_layout_passes=False),
        scratch_types=(
            [pltpu.VMEM((3 * _C,), jnp.float32)]
            + [pltpu.VMEM((3 * _L, _C), jnp.float32) for _ in range(2)]
            + [pltpu.VMEM((_BPS,), jnp.int32) for _ in range(2 * _M)]
            + [pltpu.VMEM((_BPS,), jnp.int32) for _ in range(2 * _M)]
            + [pltpu.VMEM((2 * _L, _C), jnp.float32),
               pltpu.SemaphoreType.DMA, pltpu.SemaphoreType.DMA]
        ),
    )
    return f(xflat, tbl)


_PBLK = 64
_NBLK = _L * (_T // 128)
_BPW = _NBLK // _NW
_NPB = _BPW // _PBLK


def _pack_body(tblv_hbm, out_hbm, inv, outv, sem):
    wid = lax.axis_index("s") * _NC + lax.axis_index("c")
    b0 = wid * _BPW

    def it(k, carry):
        blk = b0 + k * _PBLK
        pltpu.sync_copy(tblv_hbm.at[pl.ds(blk * 256, _PBLK * 256)], inv)

        def grp(j, c2):
            jb = j >> 3
            g = j & 7
            off_in = jb * 256 + g * 16
            f0 = inv[pl.ds(off_in, 16)]
            f1 = inv[pl.ds(off_in + 128, 16)]
            pk = plsc.pack(f0, f1, format=plsc.PackFormat.INTERLEAVED)
            outv[pl.ds(jb * 128 + g * 16, 16)] = plsc.bitcast(pk, jnp.int32)
            return c2

        lax.fori_loop(0, _PBLK * 8, grp, 0)
        pltpu.sync_copy(outv, out_hbm.at[pl.ds(blk * 128, _PBLK * 128)])
        return carry

    lax.fori_loop(0, _NPB, it, 0)


def _pack(tblv):
    mesh = plsc.VectorSubcoreMesh(core_axis_name="c", subcore_axis_name="s")
    f = pl.kernel(
        _pack_body,
        out_type=jax.ShapeDtypeStruct((_L * _T,), jnp.int32),
        mesh=mesh,
        compiler_params=pltpu.CompilerParams(needs_layout_passes=False),
        scratch_types=[
            pltpu.VMEM((_PBLK * 256,), jnp.float32),
            pltpu.VMEM((_PBLK * 128,), jnp.int32),
            pltpu.SemaphoreType.DMA,
        ],
    )
    return f(tblv)


_KB = 16


def _mlp_body(f_ref, w1_ref, b1_ref, w2_ref, b2_ref, o_ref):
    w1 = w1_ref[...]
    b1 = b1_ref[...]
    w2 = w2_ref[...]
    b2 = b2_ref[0, 0]
    fwide = jnp.concatenate([f_ref[j] for j in range(_KB)], axis=1)
    h = lax.dot_general(w1, fwide, (((0,), (0,)), ((), ())),
                        preferred_element_type=jnp.float32)
    h = jnp.maximum(h + b1, 0.0)
    res = jnp.sum(h * w2, axis=0) + b2
    for j in range(_KB):
        o_ref[j, 0, :] = res[j * _C:(j + 1) * _C]


def _mlp(feats, W1, b1, W2, b2):
    grid = (_TOTCH // _KB,)
    out = pl.pallas_call(
        _mlp_body,
        grid=grid,
        in_specs=[
            pl.BlockSpec((_KB, 2 * _L, _C), lambda i: (i, 0, 0)),
            pl.BlockSpec((2 * _L, _HIDDEN), lambda i: (0, 0)),
            pl.BlockSpec((_HIDDEN, 1), lambda i: (0, 0)),
            pl.BlockSpec((_HIDDEN, 1), lambda i: (0, 0)),
            pl.BlockSpec((1, 1), lambda i: (0, 0)),
        ],
        out_specs=pl.BlockSpec((_KB, 1, _C), lambda i: (i, 0, 0)),
        out_shape=jax.ShapeDtypeStruct((_TOTCH, 1, _C), jnp.float32),
    )(feats, W1, b1.reshape(_HIDDEN, 1), W2.reshape(_HIDDEN, 1),
      b2.reshape(1, 1))
    return out.reshape(_N // 2)


def kernel(x, tables, W1, b1, W2, b2):
    xflat = x.reshape(3 * _N)
    tblv = tables.reshape(_L, _T // 128, 128, _F).transpose(0, 1, 3, 2)
    tblv = tblv.reshape(_L * _T * _F)
    tbl = _pack(tblv)
    feats0 = _encode(xflat, tbl, 0)
    feats1 = _encode(xflat, tbl, 1)
    out0 = _mlp(feats0, W1, b1, W2, b2)
    out1 = _mlp(feats1, W1, b1, W2, b2)
    return jnp.concatenate([out0, out1])


# speedup vs baseline: 8.6529x; 1.0002x over previous
import jax
import jax.numpy as jnp
import numpy as np
from jax import lax
from jax.experimental import pallas as pl
from jax.experimental.pallas import tpu as pltpu
from jax.experimental.pallas import tpu_sc as plsc

_N = 262144
_L = 16
_F = 2
_T = 2 ** 19
_MASK = _T - 1
_HIDDEN = 64

_bfac = np.exp((np.log(2048.0) - np.log(16.0)) / (_L - 1))
_RES = [float(r) for r in
        np.floor(16.0 * _bfac ** np.arange(_L)).astype(np.int64)]
_P2 = int(np.uint32(2654435761).astype(np.int32))
_P3 = int(np.uint32(805459861).astype(np.int32))

_NC = 2
_NS = 16
_NW = _NC * _NS
_PPW = _N // 2 // _NW
_C = 128
_M = 4
_SUB = _C // _M
_BPS = _L * 8 * _SUB
_NCHUNK = _PPW // _C
_TOTCH = _N // 2 // _C

_CORNERS = [(dx, dy, dz) for dx in (0, 1) for dy in (0, 1) for dz in (0, 1)]

_GDN = lax.GatherDimensionNumbers(offset_dims=(), collapsed_slice_dims=(0,),
                                  start_index_map=(0,))
_SRCBOUNDS = {0: (6, 11), 1: (5, 11), 2: (5, 10)}


def _deinterleave(a, b, c, lane):
    out = []
    for d in range(3):
        lo, hi = _SRCBOUNDS[d]
        pidx = ((lane * 3 + d) & 15)[:, None]
        tk = lambda v: lax.gather(v, pidx, _GDN, (1,),
                                  mode=lax.GatherScatterMode.PROMISE_IN_BOUNDS)
        v = jnp.where(lane < lo, tk(a), jnp.where(lane < hi, tk(b), tk(c)))
        out.append(v)
    return out


def _encode_body(half, x_hbm, tbl_hbm, out_hbm, xv, fracvs, idxs, rows,
                 featsv, sems):
    wid = lax.axis_index("s") * _NC + lax.axis_index("c")
    base_w = half * (_N // 2) + wid * _PPW
    lane = lax.iota(jnp.int32, 16)

    def stage_a(ci, par):
        base = base_w + ci * _C
        pltpu.sync_copy(x_hbm.at[pl.ds(3 * base, 3 * _C)], xv)
        fracv = fracvs[par]
        for m in range(_M):
            def hash_group(g, carry2, m=m):
                off = m * _SUB + g * 16
                soff = g * 16
                a = xv[pl.ds(3 * off, 16)]
                b = xv[pl.ds(3 * off + 16, 16)]
                c3 = xv[pl.ds(3 * off + 32, 16)]
                xs = _deinterleave(a, b, c3, lane)
                xs = [jnp.minimum(jnp.maximum((v + 1.0) * 0.5, 0.0),
                                  1.0 - 1e-6) for v in xs]
                for l in range(_L):
                    r = _RES[l]
                    px, py, pz = xs[0] * r, xs[1] * r, xs[2] * r
                    ix = px.astype(jnp.int32)
                    iy = py.astype(jnp.int32)
                    iz = pz.astype(jnp.int32)
                    fracv[3 * l + 0, pl.ds(off, 16)] = px - ix.astype(jnp.float32)
                    fracv[3 * l + 1, pl.ds(off, 16)] = py - iy.astype(jnp.float32)
                    fracv[3 * l + 2, pl.ds(off, 16)] = pz - iz.astype(jnp.float32)
                    hx = (ix, ix + 1)
                    hy = (iy * _P2, (iy + 1) * _P2)
                    hz = (iz * _P3, (iz + 1) * _P3)
                    for c, (dx, dy, dz) in enumerate(_CORNERS):
                        s = (l * 8 + c) * _SUB + soff
                        e = ((hx[dx] ^ hy[dy] ^ hz[dz]) & _MASK) + l * _T
                        idxs[par * _M + m][pl.ds(s, 16)] = e
                return carry2

            lax.fori_loop(0, _SUB // 16, hash_group, 0)
            pltpu.async_copy(tbl_hbm.at[idxs[par * _M + m]],
                             rows[par * _M + m], sems[par])

    def stage_wait(par):
        for m in range(_M):
            pltpu.make_async_copy(tbl_hbm.at[idxs[par * _M + m]],
                                  rows[par * _M + m], sems[par]).wait()

    def stage_c(ci, par):
        base = base_w + ci * _C
        chunkid = wid * _NCHUNK + ci
        fracv = fracvs[par]
        for m in range(_M):
            def interp_group(g, carry2, m=m):
                off = m * _SUB + g * 16
                soff = g * 16
                for l in range(_L):
                    fx = fracv[3 * l + 0, pl.ds(off, 16)]
                    fy = fracv[3 * l + 1, pl.ds(off, 16)]
                    fz = fracv[3 * l + 2, pl.ds(off, 16)]
                    wx = (1.0 - fx, fx)
                    wy = (1.0 - fy, fy)
                    wz = (1.0 - fz, fz)
                    acc0 = jnp.zeros((16,), jnp.float32)
                    acc1 = jnp.zeros((16,), jnp.float32)
                    for c, (dx, dy, dz) in enumerate(_CORNERS):
                        s = (l * 8 + c) * _SUB + soff
                        v = rows[par * _M + m][pl.ds(s, 16)]
                        f0 = plsc.bitcast(v << 16, jnp.float32)
                        f1 = plsc.bitcast(v & jnp.int32(-65536), jnp.float32)
                        w = wx[dx] * wy[dy] * wz[dz]
                        acc0 = acc0 + w * f0
                        acc1 = acc1 + w * f1
                    featsv[2 * l, pl.ds(off, 16)] = acc0
                    featsv[2 * l + 1, pl.ds(off, 16)] = acc1
                return carry2

            lax.fori_loop(0, _SUB // 16, interp_group, 0)

        pltpu.sync_copy(featsv, out_hbm.at[chunkid])

    stage_a(0, 0)

    def body(k2, carry):
        c0 = 2 * k2
        stage_a(c0 + 1, 1)
        stage_wait(0)
        stage_c(c0, 0)

        @pl.when(c0 + 2 < _NCHUNK)
        def _prefetch():
            stage_a(c0 + 2, 0)

        stage_wait(1)
        stage_c(c0 + 1, 1)
        return carry

    lax.fori_loop(0, _NCHUNK // 2, body, 0)


def _make_encode_entry(half):
    def _encode_entry(x_hbm, tbl_hbm, out_hbm, xv, f0v, f1v, *rest):
        idxs = list(rest[:2 * _M])
        rows = list(rest[2 * _M:4 * _M])
        featsv = rest[4 * _M]
        sems = list(rest[4 * _M + 1:4 * _M + 3])
        _encode_body(half, x_hbm, tbl_hbm, out_hbm, xv, [f0v, f1v], idxs,
                     rows, featsv, sems)
    return _encode_entry


def _encode(xflat, tbl, half):
    mesh = plsc.VectorSubcoreMesh(core_axis_name="c", subcore_axis_name="s")
    f = pl.kernel(
        _make_encode_entry(half),
        out_type=jax.ShapeDtypeStruct((_TOTCH, 2 * _L, _C), jnp.float32),
        mesh=mesh,
        compiler_params=pltpu.CompilerParams(needs_layout_passes=False),
        scratch_types=(
            [pltpu.VMEM((3 * _C,), jnp.float32)]
            + [pltpu.VMEM((3 * _L, _C), jnp.float32) for _ in range(2)]
            + [pltpu.VMEM((_BPS,), jnp.int32) for _ in range(2 * _M)]
            + [pltpu.VMEM((_BPS,), jnp.int32) for _ in range(2 * _M)]
            + [pltpu.VMEM((2 * _L, _C), jnp.float32),
               pltpu.SemaphoreType.DMA, pltpu.SemaphoreType.DMA]
        ),
    )
    return f(xflat, tbl)


_PBLK = 64
_NBLK = _L * (_T // 128)
_BPW = _NBLK // _NW
_NPB = _BPW // _PBLK


def _pack_body(tblv_hbm, out_hbm, inv, outv, sem):
    wid = lax.axis_index("s") * _NC + lax.axis_index("c")
    b0 = wid * _BPW

    def it(k, carry):
        blk = b0 + k * _PBLK
        pltpu.sync_copy(tblv_hbm.at[pl.ds(blk * 256, _PBLK * 256)], inv)

        def grp(j, c2):
            jb = j >> 3
            g = j & 7
            off_in = jb * 256 + g * 16
            f0 = inv[pl.ds(off_in, 16)]
            f1 = inv[pl.ds(off_in + 128, 16)]
            pk = plsc.pack(f0, f1, format=plsc.PackFormat.INTERLEAVED)
            outv[pl.ds(jb * 128 + g * 16, 16)] = plsc.bitcast(pk, jnp.int32)
            return c2

        lax.fori_loop(0, _PBLK * 8, grp, 0)
        pltpu.sync_copy(outv, out_hbm.at[pl.ds(blk * 128, _PBLK * 128)])
        return carry

    lax.fori_loop(0, _NPB, it, 0)


def _pack(tblv):
    mesh = plsc.VectorSubcoreMesh(core_axis_name="c", subcore_axis_name="s")
    f = pl.kernel(
        _pack_body,
        out_type=jax.ShapeDtypeStruct((_L * _T,), jnp.int32),
        mesh=mesh,
        compiler_params=pltpu.CompilerParams(needs_layout_passes=False),
        scratch_types=[
            pltpu.VMEM((_PBLK * 256,), jnp.float32),
            pltpu.VMEM((_PBLK * 128,), jnp.int32),
            pltpu.SemaphoreType.DMA,
        ],
    )
    return f(tblv)


_KB = 16


def _mlp_body(f_ref, w1_ref, b1_ref, w2_ref, b2_ref, o_ref):
    w1 = w1_ref[...]
    b1 = b1_ref[...]
    w2 = w2_ref[...]
    b2 = b2_ref[0, 0]
    fwide = jnp.concatenate([f_ref[j] for j in range(_KB)], axis=1)
    h = lax.dot_general(w1, fwide, (((0,), (0,)), ((), ())),
                        preferred_element_type=jnp.float32)
    h = jnp.maximum(h + b1, 0.0)
    res = jnp.sum(h * w2, axis=0) + b2
    for j in range(_KB):
        o_ref[j, 0, :] = res[j * _C:(j + 1) * _C]


def _mlp(feats, W1, b1, W2, b2):
    grid = (_TOTCH // _KB,)
    out = pl.pallas_call(
        _mlp_body,
        grid=grid,
        in_specs=[
            pl.BlockSpec((_KB, 2 * _L, _C), lambda i: (i, 0, 0)),
            pl.BlockSpec((2 * _L, _HIDDEN), lambda i: (0, 0)),
            pl.BlockSpec((_HIDDEN, 1), lambda i: (0, 0)),
            pl.BlockSpec((_HIDDEN, 1), lambda i: (0, 0)),
            pl.BlockSpec((1, 1), lambda i: (0, 0)),
        ],
        out_specs=pl.BlockSpec((_KB, 1, _C), lambda i: (i, 0, 0)),
        out_shape=jax.ShapeDtypeStruct((_TOTCH, 1, _C), jnp.float32),
    )(feats, W1, b1.reshape(_HIDDEN, 1), W2.reshape(_HIDDEN, 1),
      b2.reshape(1, 1))
    return out.reshape(_N // 2)


def kernel(x, tables, W1, b1, W2, b2):
    xflat = x.reshape(3 * _N)
    tblv = tables.reshape(_L, _T // 128, 128, _F).transpose(0, 1, 3, 2)
    tblv = tblv.reshape(_L * _T * _F)
    tbl = _pack(tblv)
    feats0 = _encode(xflat, tbl, 0)
    feats1 = _encode(xflat, tbl, 1)
    out0 = _mlp(feats0, W1, b1, W2, b2)
    out1 = _mlp(feats1, W1, b1, W2, b2)
    return jnp.concatenate([out0, out1])
